# baseline (device time: 80281 ns/iter reference)
import jax
import jax.numpy as jnp
from jax import lax
from jax.experimental import pallas as pl
from jax.experimental.pallas import tpu as pltpu

B, H, D, BS = 16, 16, 64, 16
NPAGES_LOCAL = 128
NKEYS = NPAGES_LOCAL * BS
NSLOTS = 128
SCALE = D ** -0.5
NEG = -1e30


def _body(q_ref, k_ref, v_ref, bt_ref, lens_ref, out_ref,
          m_snd, l_snd, o_snd, m_rcv, l_rcv, o_rcv,
          send_sems, recv_sems):
    my_x = lax.axis_index("x")
    my_y = lax.axis_index("y")
    peer = (my_x, 1 - my_y)

    barrier = pltpu.get_barrier_semaphore()
    pl.semaphore_signal(barrier, inc=1, device_id=peer,
                        device_id_type=pl.DeviceIdType.MESH)
    pl.semaphore_wait(barrier, 1)

    bt = bt_ref[...] - my_y * NPAGES_LOCAL
    iota_j = lax.broadcasted_iota(jnp.int32, (B, NSLOTS), 1)
    valid = iota_j < lens_ref[...]
    btv = jnp.where(valid, bt, -1)
    iota_p = lax.broadcasted_iota(jnp.int32, (B, NPAGES_LOCAL, NSLOTS), 1)
    hits = btv[:, None, :] == iota_p
    counts = jnp.sum(hits.astype(jnp.float32), axis=2)

    expand = (
        lax.broadcasted_iota(jnp.int32, (NPAGES_LOCAL, NKEYS), 0)
        == lax.broadcasted_iota(jnp.int32, (NPAGES_LOCAL, NKEYS), 1) // BS
    ).astype(jnp.float32)
    w = jax.lax.dot_general(counts, expand, (((1,), (0,)), ((), ())),
                            preferred_element_type=jnp.float32)
    wpos = w > 0.0

    for h in range(H):
        qh = q_ref[:, 0, h, :].astype(jnp.bfloat16)
        kh = k_ref[:, :, h, :].reshape(NKEYS, D).astype(jnp.bfloat16)
        s = jax.lax.dot_general(qh, kh, (((1,), (1,)), ((), ())),
                                preferred_element_type=jnp.float32) * SCALE
        s = jnp.where(wpos, s, NEG)
        m_h = jnp.max(s, axis=1, keepdims=True)
        e = w * jnp.exp(s - m_h)
        l_h = jnp.sum(e, axis=1, keepdims=True)
        vh = v_ref[:, :, h, :].reshape(NKEYS, D).astype(jnp.bfloat16)
        o_h = jax.lax.dot_general(e.astype(jnp.bfloat16), vh,
                                  (((1,), (0,)), ((), ())),
                                  preferred_element_type=jnp.float32)
        m_snd[:, h:h + 1] = m_h
        l_snd[:, h:h + 1] = l_h
        o_snd[:, h, :] = o_h

    copies = [
        pltpu.make_async_remote_copy(
            src_ref=src, dst_ref=dst,
            send_sem=send_sems.at[i], recv_sem=recv_sems.at[i],
            device_id=peer, device_id_type=pl.DeviceIdType.MESH)
        for i, (src, dst) in enumerate(
            [(m_snd, m_rcv), (l_snd, l_rcv), (o_snd, o_rcv)])
    ]
    for c in copies:
        c.start()
    for c in copies:
        c.wait()

    m_s, m_r = m_snd[...], m_rcv[...]
    m_n = jnp.maximum(m_s, m_r)
    a_s = jnp.exp(m_s - m_n)
    a_r = jnp.exp(m_r - m_n)
    l_n = l_snd[...] * a_s + l_rcv[...] * a_r
    for h in range(H):
        o_h = (o_snd[:, h, :] * a_s[:, h:h + 1]
               + o_rcv[:, h, :] * a_r[:, h:h + 1])
        out_ref[:, 0, h, :] = o_h / l_n[:, h:h + 1]


def kernel(Q, K, V, bt, lens):
    lens2 = lens.reshape(B, 1)
    return pl.pallas_call(
        _body,
        out_shape=jax.ShapeDtypeStruct((B, 1, H, D), jnp.float32),
        in_specs=[pl.BlockSpec(memory_space=pltpu.VMEM)] * 5,
        out_specs=pl.BlockSpec(memory_space=pltpu.VMEM),
        scratch_shapes=[
            pltpu.VMEM((B, H), jnp.float32),
            pltpu.VMEM((B, H), jnp.float32),
            pltpu.VMEM((B, H, D), jnp.float32),
            pltpu.VMEM((B, H), jnp.float32),
            pltpu.VMEM((B, H), jnp.float32),
            pltpu.VMEM((B, H, D), jnp.float32),
            pltpu.SemaphoreType.DMA((3,)),
            pltpu.SemaphoreType.DMA((3,)),
        ],
        compiler_params=pltpu.CompilerParams(collective_id=0),
    )(Q, K, V, bt, lens2)


# device time: 20962 ns/iter; 3.8298x vs baseline; 3.8298x over previous
import jax
import jax.numpy as jnp
from jax import lax
from jax.experimental import pallas as pl
from jax.experimental.pallas import tpu as pltpu

B, H, D, BS = 16, 16, 64, 16
NPAGES_LOCAL = 128
NKEYS = NPAGES_LOCAL * BS
NSLOTS = 128
HB = H * B
HD = H * D
SCALE = D ** -0.5
NEG = -1e30


def _body(q_ref, kp_ref, vp_ref, bt_ref, lens_ref, out_ref,
          l_scr, m_snd, l_snd, o_snd, m_rcv, l_rcv, o_rcv,
          send_sems, recv_sems):
    my_x = lax.axis_index("x")
    my_y = lax.axis_index("y")
    peer = (my_x, 1 - my_y)

    barrier = pltpu.get_barrier_semaphore()
    pl.semaphore_signal(barrier, inc=1, device_id=peer,
                        device_id_type=pl.DeviceIdType.MESH)
    pl.semaphore_wait(barrier, 1)

    bt = bt_ref[...] - my_y * NPAGES_LOCAL
    iota_j = lax.broadcasted_iota(jnp.int32, (B, NSLOTS), 1)
    valid = iota_j < lens_ref[...]
    btv = jnp.where(valid, bt, -1)
    iota_p = lax.broadcasted_iota(jnp.int32, (B, NPAGES_LOCAL, NSLOTS), 1)
    hits = btv[:, None, :] == iota_p
    counts = jnp.sum(hits.astype(jnp.float32), axis=2)

    w = jnp.concatenate([counts] * BS, axis=1)
    w_hb = jnp.concatenate([w] * H, axis=0)
    wpos = w_hb > 0.0

    l_scr[...] = jnp.zeros((HB, HD), jnp.bfloat16)
    for h in range(H):
        l_scr[h * B:(h + 1) * B, h * D:(h + 1) * D] = (
            q_ref[:, 0, h, :].astype(jnp.bfloat16))
    lhs = l_scr[...]

    s_parts = []
    for s in range(BS):
        ks = kp_ref[s].reshape(HD, NPAGES_LOCAL).astype(jnp.bfloat16)
        s_parts.append(jax.lax.dot_general(
            lhs, ks, (((1,), (0,)), ((), ())),
            preferred_element_type=jnp.float32))
    s_all = jnp.concatenate(s_parts, axis=1) * SCALE

    s_all = jnp.where(wpos, s_all, NEG)
    m = jnp.max(s_all, axis=1, keepdims=True)
    e = w_hb * jnp.exp(s_all - m)
    l = jnp.sum(e, axis=1, keepdims=True)

    eb = e.astype(jnp.bfloat16)
    o_full = jnp.zeros((HB, HD), jnp.float32)
    for s in range(BS):
        vs = vp_ref[s].reshape(HD, NPAGES_LOCAL).astype(jnp.bfloat16)
        o_full = o_full + jax.lax.dot_general(
            eb[:, s * NPAGES_LOCAL:(s + 1) * NPAGES_LOCAL], vs,
            (((1,), (1,)), ((), ())),
            preferred_element_type=jnp.float32)

    for h in range(H):
        rows = slice(h * B, (h + 1) * B)
        m_snd[:, h:h + 1] = m[rows, :]
        l_snd[:, h:h + 1] = l[rows, :]
        o_snd[:, h, :] = o_full[rows, h * D:(h + 1) * D]

    copies = [
        pltpu.make_async_remote_copy(
            src_ref=src, dst_ref=dst,
            send_sem=send_sems.at[i], recv_sem=recv_sems.at[i],
            device_id=peer, device_id_type=pl.DeviceIdType.MESH)
        for i, (src, dst) in enumerate(
            [(m_snd, m_rcv), (l_snd, l_rcv), (o_snd, o_rcv)])
    ]
    for c in copies:
        c.start()
    for c in copies:
        c.wait()

    m_s, m_r = m_snd[...], m_rcv[...]
    m_n = jnp.maximum(m_s, m_r)
    a_s = jnp.exp(m_s - m_n)
    a_r = jnp.exp(m_r - m_n)
    l_n = l_snd[...] * a_s + l_rcv[...] * a_r
    for h in range(H):
        o_h = (o_snd[:, h, :] * a_s[:, h:h + 1]
               + o_rcv[:, h, :] * a_r[:, h:h + 1])
        out_ref[:, 0, h, :] = o_h / l_n[:, h:h + 1]


def kernel(Q, K, V, bt, lens):
    kp = jnp.transpose(K, (1, 2, 3, 0))
    vp = jnp.transpose(V, (1, 2, 3, 0))
    lens2 = lens.reshape(B, 1)
    return pl.pallas_call(
        _body,
        out_shape=jax.ShapeDtypeStruct((B, 1, H, D), jnp.float32),
        in_specs=[pl.BlockSpec(memory_space=pltpu.VMEM)] * 5,
        out_specs=pl.BlockSpec(memory_space=pltpu.VMEM),
        scratch_shapes=[
            pltpu.VMEM((HB, HD), jnp.bfloat16),
            pltpu.VMEM((B, H), jnp.float32),
            pltpu.VMEM((B, H), jnp.float32),
            pltpu.VMEM((B, H, D), jnp.float32),
            pltpu.VMEM((B, H), jnp.float32),
            pltpu.VMEM((B, H), jnp.float32),
            pltpu.VMEM((B, H, D), jnp.float32),
            pltpu.SemaphoreType.DMA((3,)),
            pltpu.SemaphoreType.DMA((3,)),
        ],
        compiler_params=pltpu.CompilerParams(collective_id=0),
    )(Q, kp, vp, bt, lens2)
